# baseline (speedup 1.0000x reference)
import jax
import jax.numpy as jnp
from jax.experimental import pallas as pl
from jax.experimental.pallas import tpu as pltpu


def _pw_conv_kernel(x_ref, w_ref, b_ref, o_ref):
    xb = x_ref[0].astype(jnp.bfloat16)
    acc = jnp.dot(w_ref[...], xb, preferred_element_type=jnp.float32)
    o_ref[0] = acc + b_ref[...]


def kernel(x, weight, bias):
    N, C_in, L = x.shape
    C_out = weight.shape[0]

    w_bf = weight[:, :, 0].astype(jnp.bfloat16)
    b_2d = bias.reshape(C_out, 1).astype(jnp.float32)

    TL = 2048
    if L <= TL:
        TL, num_l = L, 1
    else:
        num_l = pl.cdiv(L, TL)

    TCO = 512
    num_co = C_out // TCO if C_out % TCO == 0 and C_out > TCO else 1
    TCO = C_out // num_co

    itemsize = jnp.dtype(x.dtype).itemsize
    cost = pl.CostEstimate(
        flops=2 * N * L * C_in * C_out,
        transcendentals=0,
        bytes_accessed=(N * C_in * L + N * C_out * L) * itemsize
        + C_out * C_in * 2 + C_out * 4,
    )

    return pl.pallas_call(
        _pw_conv_kernel,
        out_shape=jax.ShapeDtypeStruct((N, C_out, L), x.dtype),
        grid=(N * num_l, num_co),
        in_specs=[
            pl.BlockSpec((1, C_in, TL),
                         lambda i, co: (i // num_l, 0, i % num_l)),
            pl.BlockSpec((TCO, C_in), lambda i, co: (co, 0)),
            pl.BlockSpec((TCO, 1), lambda i, co: (co, 0)),
        ],
        out_specs=pl.BlockSpec((1, TCO, TL),
                               lambda i, co: (i // num_l, co, i % num_l)),
        compiler_params=pltpu.CompilerParams(
            dimension_semantics=("parallel", "arbitrary")),
        cost_estimate=cost,
    )(x, w_bf, b_2d)


# speedup vs baseline: 1.3940x; 1.3940x over previous
import jax
import jax.numpy as jnp
from jax.experimental import pallas as pl
from jax.experimental.pallas import tpu as pltpu


def _pw_conv_kernel(x_ref, w_ref, b_ref, o_ref):
    xb = x_ref[0].astype(jnp.bfloat16)
    acc = jnp.dot(w_ref[...], xb, preferred_element_type=jnp.float32)
    o_ref[0] = acc + b_ref[...]


def kernel(x, weight, bias):
    N, C_in, L = x.shape
    C_out = weight.shape[0]

    w_bf = weight[:, :, 0].astype(jnp.bfloat16)
    b_2d = bias.reshape(C_out, 1).astype(jnp.float32)

    TL = 2048
    if L <= TL:
        TL, num_l = L, 1
    else:
        num_l = pl.cdiv(L, TL)

    itemsize = jnp.dtype(x.dtype).itemsize
    cost = pl.CostEstimate(
        flops=2 * N * L * C_in * C_out,
        transcendentals=0,
        bytes_accessed=(N * C_in * L + N * C_out * L) * itemsize
        + C_out * C_in * 2 + C_out * 4,
    )

    return pl.pallas_call(
        _pw_conv_kernel,
        out_shape=jax.ShapeDtypeStruct((N, C_out, L), x.dtype),
        grid=(N * num_l,),
        in_specs=[
            pl.BlockSpec((1, C_in, TL), lambda i: (i // num_l, 0, i % num_l)),
            pl.BlockSpec((C_out, C_in), lambda i: (0, 0)),
            pl.BlockSpec((C_out, 1), lambda i: (0, 0)),
        ],
        out_specs=pl.BlockSpec((1, C_out, TL),
                               lambda i: (i // num_l, 0, i % num_l)),
        compiler_params=pltpu.CompilerParams(dimension_semantics=("parallel",)),
        cost_estimate=cost,
    )(x, w_bf, b_2d)
